# Initial kernel scaffold; baseline (speedup 1.0000x reference)
import jax
import jax.numpy as jnp
from jax.experimental import pallas as pl


def kernel(x, edge_index, e, xbatch, bn_node_gamma, bn_node_beta, bn_edge_gamma, bn_edge_beta, w1, b1, w2, b2, w3, b3, w4, b4, w5, b5, w6, b6):
    raise NotImplementedError("write your pallas kernel here")


# speedup vs baseline: 2.0553x; 2.0553x over previous
import functools
import jax
import jax.numpy as jnp
from jax import lax
from jax.experimental import pallas as pl
from jax.experimental.pallas import tpu as pltpu

LEAK = 0.1
BN_EPS = 1e-5
TILE_E = 1024
HALF_E = 512
TILE_P = 2048
GROUP = 4


def _round_up(a, b):
    return (a + b - 1) // b * b


def _dot(a, b):
    return jnp.dot(a, b, preferred_element_type=jnp.float32)


def _dot_tb(a, b):
    return lax.dot_general(a, b, (((1,), (1,)), ((), ())),
                           preferred_element_type=jnp.float32)


def _dot_ta_tb(a, b):
    return lax.dot_general(a, b, (((0,), (1,)), ((), ())),
                           preferred_element_type=jnp.float32)


def _leaky(h):
    return jnp.where(h > 0, h, h * LEAK)


def _stats_kernel(e_ref, out_ref, acc_ref):
    t = pl.program_id(1)

    @pl.when(t == 0)
    def _():
        acc_ref[...] = jnp.zeros_like(acc_ref)

    e = e_ref[...]
    acc_ref[0:1, :] = acc_ref[0:1, :] + jnp.sum(e, axis=0, keepdims=True)
    acc_ref[1:2, :] = acc_ref[1:2, :] + jnp.sum(e * e, axis=0, keepdims=True)

    @pl.when(t == pl.num_programs(1) - 1)
    def _():
        out_ref[...] = acc_ref[...].reshape(1, 2, -1)


def _edge_stats(e_pad, n_ef, t2):
    return pl.pallas_call(
        _stats_kernel,
        grid=(2, t2),
        in_specs=[pl.BlockSpec((TILE_P, n_ef), lambda c, t: (c * t2 + t, 0))],
        out_specs=pl.BlockSpec((1, 2, n_ef), lambda c, t: (c, 0, 0)),
        out_shape=jax.ShapeDtypeStruct((2, 2, n_ef), jnp.float32),
        scratch_shapes=[pltpu.VMEM((2, n_ef), jnp.float32)],
        compiler_params=pltpu.CompilerParams(
            dimension_semantics=("parallel", "arbitrary")),
    )(e_pad)


def _main_kernel(esums_ref, xg_ref, gx_ref, bx_ref, ge_ref, be_ref,
                 w4s_ref, w4d_ref, w1et_ref, b1_ref,
                 w2a_ref, w3a_ref, w4a_ref, w5a_ref, w6a_ref,
                 e_ref, idx_ref, out_ref,
                 agt_ref, bgt_ref, w1ee_ref, b1cb_ref,
                 *, n_edges, n_nf, n_ef, n_groups):
    t = pl.program_id(1)

    @pl.when(t == 0)
    def _():
        p = esums_ref[...]
        s = p[0] + p[1]
        inv_n = jnp.float32(1.0 / n_edges)
        mean_e = s[0:1, :] * inv_n
        var_e = s[1:2, :] * inv_n - mean_e * mean_e
        scale_e = ge_ref[...] * lax.rsqrt(var_e + BN_EPS)
        shift_e = be_ref[...] - mean_e * scale_e
        w1ee_ref[...] = w1et_ref[...] * scale_e
        b1_eff = b1_ref[...] + _dot_tb(shift_e, w1et_ref[...])
        ones_row = jnp.ones((1, HALF_E), jnp.float32)
        b1cb_ref[...] = lax.dot_general(
            b1_eff, ones_row, (((0,), (0,)), ((), ())),
            preferred_element_type=jnp.float32)

        xg = xg_ref[...]
        s4 = jnp.mean(xg, axis=0, keepdims=True)
        ss4 = jnp.mean(xg * xg, axis=0, keepdims=True)
        m = jnp.zeros((1, n_nf), jnp.float32)
        msq = jnp.zeros((1, n_nf), jnp.float32)
        for k in range(GROUP):
            m = m + s4[:, k * n_nf:(k + 1) * n_nf]
            msq = msq + ss4[:, k * n_nf:(k + 1) * n_nf]
        m = m * (1.0 / GROUP)
        msq = msq * (1.0 / GROUP)
        var_n = msq - m * m
        scale_n = gx_ref[...] * lax.rsqrt(var_n + BN_EPS)
        shift_n = bx_ref[...] - m * scale_n
        scale4 = jnp.concatenate([scale_n] * GROUP, axis=1)
        shift4 = jnp.concatenate([shift_n] * GROUP, axis=1)
        xn = xg * scale4 + shift4
        agt_ref[...] = _dot_ta_tb(w4s_ref[...], xn)
        bgt_ref[...] = _dot_ta_tb(w4d_ref[...], xn)

    agt = agt_ref[...]
    bgt = bgt_ref[...]
    w1ee = w1ee_ref[...]
    b1cb = b1cb_ref[...]
    ones_row = jnp.ones((1, HALF_E), jnp.float32)
    iota_g = lax.broadcasted_iota(jnp.int32, (n_groups, HALF_E), 0)

    for h in range(TILE_E // HALF_E):
        sl = slice(h * HALF_E, (h + 1) * HALF_E)
        src = idx_ref[0:1, sl]
        dst = idx_ref[1:2, sl]
        ms = jnp.where(iota_g == (src >> 2), 1.0, 0.0)
        md = jnp.where(iota_g == (dst >> 2), 1.0, 0.0)
        gs = _dot(agt, ms)
        gd = _dot(bgt, md)
        srcr = src & 3
        dstr = dst & 3
        h1 = b1cb
        for k in range(GROUP):
            fs = jnp.where(srcr == k, 1.0, 0.0)
            fd = jnp.where(dstr == k, 1.0, 0.0)
            h1 = h1 + fs * gs[k * 64:(k + 1) * 64, :]
            h1 = h1 + fd * gd[k * 64:(k + 1) * 64, :]
        h1 = h1 + _dot_tb(w1ee, e_ref[sl, :])
        hcur = _leaky(h1)
        for wa_ref in (w2a_ref, w3a_ref, w4a_ref, w5a_ref):
            hh = jnp.concatenate([hcur, ones_row], axis=0)
            hcur = _leaky(_dot(wa_ref[...], hh))
        hh = jnp.concatenate([hcur, ones_row], axis=0)
        out8 = _dot(w6a_ref[...], hh)
        out_ref[:, sl] = out8[0:2, :]


def _edge_mlp(esums, xg, gx, bx, ge, be, w4s, w4d, w1et, b1,
              tails, e_pad, idx_pad, n_edges, t2):
    n_groups = xg.shape[0]
    n_nf = gx.shape[1]
    n_ef = e_pad.shape[1]
    e_rows = e_pad.shape[0]
    const = lambda c, t: (0, 0)
    small = [esums, xg, gx, bx, ge, be, w4s, w4d, w1et, b1] + list(tails)
    in_specs = (
        [pl.BlockSpec((2, 2, n_ef), lambda c, t: (0, 0, 0))]
        + [pl.BlockSpec(a.shape, const) for a in small[1:]]
        + [pl.BlockSpec((TILE_E, n_ef), lambda c, t: (c * t2 + t, 0)),
           pl.BlockSpec((2, TILE_E), lambda c, t: (0, c * t2 + t))]
    )
    mlp_flops = 2 * (64 * (2 * GROUP * 64 + n_ef) + 64 * 65 + 32 * 65
                     + 16 * 33 + 8 * 17 + 8 * 9)
    cost = pl.CostEstimate(
        flops=e_rows * mlp_flops + 2 * e_rows * n_groups * GROUP * 64,
        transcendentals=0,
        bytes_accessed=4 * e_rows * (n_ef + 4),
    )
    return pl.pallas_call(
        functools.partial(_main_kernel, n_edges=n_edges, n_nf=n_nf,
                          n_ef=n_ef, n_groups=n_groups),
        grid=(2, t2),
        in_specs=in_specs,
        out_specs=pl.BlockSpec((2, TILE_E), lambda c, t: (0, c * t2 + t)),
        out_shape=jax.ShapeDtypeStruct((2, e_rows), jnp.float32),
        scratch_shapes=[
            pltpu.VMEM((GROUP * 64, n_groups), jnp.float32),
            pltpu.VMEM((GROUP * 64, n_groups), jnp.float32),
            pltpu.VMEM((64, n_ef), jnp.float32),
            pltpu.VMEM((64, HALF_E), jnp.float32),
        ],
        compiler_params=pltpu.CompilerParams(
            dimension_semantics=("parallel", "arbitrary")),
        cost_estimate=cost,
    )(*small, e_pad, idx_pad)


def kernel(x, edge_index, e, xbatch,
           bn_node_gamma, bn_node_beta, bn_edge_gamma, bn_edge_beta,
           w1, b1, w2, b2, w3, b3, w4, b4, w5, b5, w6, b6):
    del xbatch
    f32 = jnp.float32
    n_nodes, n_nf = x.shape
    n_edges, n_ef = e.shape
    n_groups = n_nodes // GROUP

    e_rows = _round_up(max(n_edges, 1), max(2 * TILE_E, 2 * TILE_P))
    e_pad = jnp.pad(e.astype(f32), ((0, e_rows - n_edges), (0, 0)))
    idx_pad = jnp.pad(edge_index.astype(jnp.int32),
                      ((0, 0), (0, e_rows - n_edges)))

    xg = x.astype(f32).reshape(n_groups, GROUP * n_nf)
    w1s, w1d = w1[:n_nf], w1[n_nf:2 * n_nf]
    eye4 = jnp.eye(GROUP, dtype=f32)
    w4s = jnp.kron(eye4, w1s)
    w4d = jnp.kron(eye4, w1d)
    w1et = w1[2 * n_nf:].T

    def aug(w, b):
        return jnp.concatenate([w.T, b.reshape(1, -1).T], axis=1)

    w6a = jnp.zeros((8, w6.shape[0] + 1), f32).at[:w6.shape[1]].set(aug(w6, b6))
    tails = (aug(w2, b2), aug(w3, b3), aug(w4, b4), aug(w5, b5), w6a)

    esums = _edge_stats(e_pad, n_ef, e_rows // (2 * TILE_P))
    out2 = _edge_mlp(esums, xg,
                     bn_node_gamma.reshape(1, -1).astype(f32),
                     bn_node_beta.reshape(1, -1).astype(f32),
                     bn_edge_gamma.reshape(1, -1).astype(f32),
                     bn_edge_beta.reshape(1, -1).astype(f32),
                     w4s, w4d, w1et, b1.reshape(1, -1).astype(f32),
                     tails, e_pad, idx_pad, n_edges,
                     e_rows // (2 * TILE_E))
    return {'edge_pred': [out2[:, :n_edges].T]}


# speedup vs baseline: 3.5908x; 1.7471x over previous
import functools
import jax
import jax.numpy as jnp
from jax import lax
from jax.experimental import pallas as pl
from jax.experimental.pallas import tpu as pltpu

LEAK = 0.1
BN_EPS = 1e-5
TILE_E = 2048
HALF_E = 512
TILE_P = 4096
GROUP = 4


def _round_up(a, b):
    return (a + b - 1) // b * b


def _dot(a, b):
    return jnp.dot(a, b, preferred_element_type=jnp.float32)


def _dot_tb(a, b):
    return lax.dot_general(a, b, (((1,), (1,)), ((), ())),
                           preferred_element_type=jnp.float32)


def _dot_ta_tb(a, b):
    return lax.dot_general(a, b, (((0,), (1,)), ((), ())),
                           preferred_element_type=jnp.float32)


def _leaky(h):
    return jnp.where(h > 0, h, h * LEAK)


def _stats_kernel(e_ref, out_ref):
    e = e_ref[...]
    s = jnp.sum(e, axis=0, keepdims=True)
    ss = jnp.sum(e * e, axis=0, keepdims=True)
    out_ref[...] = jnp.concatenate([s, ss], axis=0).reshape(1, 2, -1)


def _edge_stats(e_pad, n_ef, n_tp):
    return pl.pallas_call(
        _stats_kernel,
        grid=(n_tp,),
        in_specs=[pl.BlockSpec((TILE_P, n_ef), lambda t: (t, 0))],
        out_specs=pl.BlockSpec((1, 2, n_ef), lambda t: (t, 0, 0)),
        out_shape=jax.ShapeDtypeStruct((n_tp, 2, n_ef), jnp.float32),
        compiler_params=pltpu.CompilerParams(
            dimension_semantics=("parallel",)),
    )(e_pad)


def _table_kernel(part_ref, xg_ref, gx_ref, bx_ref, ge_ref, be_ref,
                  w4s_ref, w4d_ref, w1et_ref, b1_ref,
                  agt_ref, bgt_ref, w1ee_ref, b1cb_ref,
                  *, n_edges, n_nf):
    s = jnp.sum(part_ref[...], axis=0)
    inv_n = jnp.float32(1.0 / n_edges)
    mean_e = s[0:1, :] * inv_n
    var_e = s[1:2, :] * inv_n - mean_e * mean_e
    scale_e = ge_ref[...] * lax.rsqrt(var_e + BN_EPS)
    shift_e = be_ref[...] - mean_e * scale_e
    w1ee_ref[...] = w1et_ref[...] * scale_e
    b1_eff = b1_ref[...] + _dot_tb(shift_e, w1et_ref[...])
    ones_row = jnp.ones((1, HALF_E), jnp.float32)
    b1cb_ref[...] = lax.dot_general(
        b1_eff, ones_row, (((0,), (0,)), ((), ())),
        preferred_element_type=jnp.float32)

    xg = xg_ref[...]
    s4 = jnp.mean(xg, axis=0, keepdims=True)
    ss4 = jnp.mean(xg * xg, axis=0, keepdims=True)
    m = jnp.zeros((1, n_nf), jnp.float32)
    msq = jnp.zeros((1, n_nf), jnp.float32)
    for k in range(GROUP):
        m = m + s4[:, k * n_nf:(k + 1) * n_nf]
        msq = msq + ss4[:, k * n_nf:(k + 1) * n_nf]
    m = m * (1.0 / GROUP)
    msq = msq * (1.0 / GROUP)
    var_n = msq - m * m
    scale_n = gx_ref[...] * lax.rsqrt(var_n + BN_EPS)
    shift_n = bx_ref[...] - m * scale_n
    scale4 = jnp.concatenate([scale_n] * GROUP, axis=1)
    shift4 = jnp.concatenate([shift_n] * GROUP, axis=1)
    xn = xg * scale4 + shift4
    agt_ref[...] = _dot_ta_tb(w4s_ref[...], xn)
    bgt_ref[...] = _dot_ta_tb(w4d_ref[...], xn)


def _build_tables(partials, xg, gx, bx, ge, be, w4s, w4d, w1et, b1, n_edges):
    n_groups = xg.shape[0]
    n_nf = gx.shape[1]
    n_ef = w1et.shape[1]
    args = [partials, xg, gx, bx, ge, be, w4s, w4d, w1et, b1]
    return pl.pallas_call(
        functools.partial(_table_kernel, n_edges=n_edges, n_nf=n_nf),
        grid=(1,),
        in_specs=[pl.BlockSpec(a.shape, lambda t, n=len(a.shape): (0,) * n)
                  for a in args],
        out_specs=[
            pl.BlockSpec((GROUP * 64, n_groups), lambda t: (0, 0)),
            pl.BlockSpec((GROUP * 64, n_groups), lambda t: (0, 0)),
            pl.BlockSpec((64, n_ef), lambda t: (0, 0)),
            pl.BlockSpec((64, HALF_E), lambda t: (0, 0)),
        ],
        out_shape=[
            jax.ShapeDtypeStruct((GROUP * 64, n_groups), jnp.float32),
            jax.ShapeDtypeStruct((GROUP * 64, n_groups), jnp.float32),
            jax.ShapeDtypeStruct((64, n_ef), jnp.float32),
            jax.ShapeDtypeStruct((64, HALF_E), jnp.float32),
        ],
        compiler_params=pltpu.CompilerParams(
            dimension_semantics=("arbitrary",)),
    )(*args)


def _main_kernel(agt_ref, bgt_ref, w1ee_ref, b1cb_ref,
                 w2a_ref, w3a_ref, w4a_ref, w5a_ref, w6a_ref,
                 e_ref, idx_ref, out_ref, *, n_groups):
    agt = agt_ref[...]
    bgt = bgt_ref[...]
    w1ee = w1ee_ref[...]
    b1cb = b1cb_ref[...]
    ones_row = jnp.ones((1, HALF_E), jnp.float32)
    iota_g = lax.broadcasted_iota(jnp.int32, (n_groups, HALF_E), 0)
    n_half = TILE_E // HALF_E
    slices = [slice(h * HALF_E, (h + 1) * HALF_E) for h in range(n_half)]

    hs = []
    for sl in slices:
        src = idx_ref[0:1, sl]
        dst = idx_ref[1:2, sl]
        ms = jnp.where(iota_g == (src >> 2), 1.0, 0.0)
        md = jnp.where(iota_g == (dst >> 2), 1.0, 0.0)
        gs = _dot(agt, ms)
        gd = _dot(bgt, md)
        srcr = src & 3
        dstr = dst & 3
        h1 = b1cb + _dot_tb(w1ee, e_ref[sl, :])
        for k in range(GROUP):
            fs = jnp.where(srcr == k, 1.0, 0.0)
            fd = jnp.where(dstr == k, 1.0, 0.0)
            h1 = h1 + fs * gs[k * 64:(k + 1) * 64, :]
            h1 = h1 + fd * gd[k * 64:(k + 1) * 64, :]
        hs.append(_leaky(h1))
    for wa_ref in (w2a_ref, w3a_ref, w4a_ref, w5a_ref):
        wa = wa_ref[...]
        hs = [_leaky(_dot(wa, jnp.concatenate([h, ones_row], axis=0)))
              for h in hs]
    w6a = w6a_ref[...]
    for h, sl in zip(hs, slices):
        out8 = _dot(w6a, jnp.concatenate([h, ones_row], axis=0))
        out_ref[:, sl] = out8[0:2, :]


def _edge_mlp(agt, bgt, w1ee, b1cb, tails, e_pad, idx_pad, n_tiles):
    n_groups = agt.shape[1]
    n_ef = e_pad.shape[1]
    e_rows = e_pad.shape[0]
    small = [agt, bgt, w1ee, b1cb] + list(tails)
    in_specs = (
        [pl.BlockSpec(a.shape, lambda t: (0, 0)) for a in small]
        + [pl.BlockSpec((TILE_E, n_ef), lambda t: (t, 0)),
           pl.BlockSpec((2, TILE_E), lambda t: (0, t))]
    )
    mlp_flops = 2 * (64 * (2 * GROUP * 64 + n_ef) + 64 * 65 + 32 * 65
                     + 16 * 33 + 8 * 17 + 8 * 9)
    cost = pl.CostEstimate(
        flops=e_rows * mlp_flops + 2 * e_rows * n_groups * GROUP * 64,
        transcendentals=0,
        bytes_accessed=4 * e_rows * (n_ef + 4),
    )
    return pl.pallas_call(
        functools.partial(_main_kernel, n_groups=n_groups),
        grid=(n_tiles,),
        in_specs=in_specs,
        out_specs=pl.BlockSpec((2, TILE_E), lambda t: (0, t)),
        out_shape=jax.ShapeDtypeStruct((2, e_rows), jnp.float32),
        compiler_params=pltpu.CompilerParams(
            dimension_semantics=("parallel",)),
        cost_estimate=cost,
    )(*small, e_pad, idx_pad)


def kernel(x, edge_index, e, xbatch,
           bn_node_gamma, bn_node_beta, bn_edge_gamma, bn_edge_beta,
           w1, b1, w2, b2, w3, b3, w4, b4, w5, b5, w6, b6):
    del xbatch
    f32 = jnp.float32
    n_nodes, n_nf = x.shape
    n_edges, n_ef = e.shape
    n_groups = n_nodes // GROUP

    e_rows = _round_up(max(n_edges, 1), max(TILE_E, TILE_P))
    e_pad = jnp.pad(e.astype(f32), ((0, e_rows - n_edges), (0, 0)))
    idx_pad = jnp.pad(edge_index.astype(jnp.int32),
                      ((0, 0), (0, e_rows - n_edges)))

    xg = x.astype(f32).reshape(n_groups, GROUP * n_nf)
    w1s, w1d = w1[:n_nf], w1[n_nf:2 * n_nf]
    eye4 = jnp.eye(GROUP, dtype=f32)
    w4s = jnp.kron(eye4, w1s)
    w4d = jnp.kron(eye4, w1d)
    w1et = w1[2 * n_nf:].T

    def aug(w, b):
        return jnp.concatenate([w.T, b.reshape(1, -1).T], axis=1)

    w6a = jnp.zeros((8, w6.shape[0] + 1), f32).at[:w6.shape[1]].set(aug(w6, b6))
    tails = (aug(w2, b2), aug(w3, b3), aug(w4, b4), aug(w5, b5), w6a)

    partials = _edge_stats(e_pad, n_ef, e_rows // TILE_P)
    agt, bgt, w1ee, b1cb = _build_tables(
        partials, xg,
        bn_node_gamma.reshape(1, -1).astype(f32),
        bn_node_beta.reshape(1, -1).astype(f32),
        bn_edge_gamma.reshape(1, -1).astype(f32),
        bn_edge_beta.reshape(1, -1).astype(f32),
        w4s, w4d, w1et, b1.reshape(1, -1).astype(f32), n_edges)
    out2 = _edge_mlp(agt, bgt, w1ee, b1cb, tails, e_pad, idx_pad,
                     e_rows // TILE_E)
    return {'edge_pred': [out2[:, :n_edges].T]}


# speedup vs baseline: 4.4863x; 1.2494x over previous
import functools
import jax
import jax.numpy as jnp
from jax import lax
from jax.experimental import pallas as pl
from jax.experimental.pallas import tpu as pltpu

LEAK = 0.1
BN_EPS = 1e-5
TILE_E = 8192
HALF_E = 512
TILE_P = 8192
GROUP = 4


def _round_up(a, b):
    return (a + b - 1) // b * b


def _dot(a, b):
    return jnp.dot(a, b, preferred_element_type=jnp.float32)


def _dot_tb(a, b):
    return lax.dot_general(a, b, (((1,), (1,)), ((), ())),
                           preferred_element_type=jnp.float32)


def _dot_ta_tb(a, b):
    return lax.dot_general(a, b, (((0,), (1,)), ((), ())),
                           preferred_element_type=jnp.float32)


def _leaky(h):
    return jnp.where(h > 0, h, h * LEAK)


def _stats_kernel(e_ref, out_ref):
    e = e_ref[...]
    s = jnp.sum(e, axis=0, keepdims=True)
    ss = jnp.sum(e * e, axis=0, keepdims=True)
    out_ref[...] = jnp.concatenate([s, ss], axis=0).reshape(1, 2, -1)


def _edge_stats(e_pad, n_ef, n_tp):
    return pl.pallas_call(
        _stats_kernel,
        grid=(n_tp,),
        in_specs=[pl.BlockSpec((TILE_P, n_ef), lambda t: (t, 0))],
        out_specs=pl.BlockSpec((1, 2, n_ef), lambda t: (t, 0, 0)),
        out_shape=jax.ShapeDtypeStruct((n_tp, 2, n_ef), jnp.float32),
        compiler_params=pltpu.CompilerParams(
            dimension_semantics=("parallel",)),
    )(e_pad)


def _table_kernel(part_ref, xg_ref, gx_ref, bx_ref, ge_ref, be_ref,
                  w4s_ref, w4d_ref, w1et_ref, b1_ref,
                  agt_ref, bgt_ref, w1ee_ref, b1cb_ref,
                  *, n_edges, n_nf):
    s = jnp.sum(part_ref[...], axis=0)
    inv_n = jnp.float32(1.0 / n_edges)
    mean_e = s[0:1, :] * inv_n
    var_e = s[1:2, :] * inv_n - mean_e * mean_e
    scale_e = ge_ref[...] * lax.rsqrt(var_e + BN_EPS)
    shift_e = be_ref[...] - mean_e * scale_e
    w1ee_ref[...] = w1et_ref[...] * scale_e
    b1_eff = b1_ref[...] + _dot_tb(shift_e, w1et_ref[...])
    ones_row = jnp.ones((1, HALF_E), jnp.float32)
    b1cb_ref[...] = lax.dot_general(
        b1_eff, ones_row, (((0,), (0,)), ((), ())),
        preferred_element_type=jnp.float32)

    xg = xg_ref[...]
    s4 = jnp.mean(xg, axis=0, keepdims=True)
    ss4 = jnp.mean(xg * xg, axis=0, keepdims=True)
    m = jnp.zeros((1, n_nf), jnp.float32)
    msq = jnp.zeros((1, n_nf), jnp.float32)
    for k in range(GROUP):
        m = m + s4[:, k * n_nf:(k + 1) * n_nf]
        msq = msq + ss4[:, k * n_nf:(k + 1) * n_nf]
    m = m * (1.0 / GROUP)
    msq = msq * (1.0 / GROUP)
    var_n = msq - m * m
    scale_n = gx_ref[...] * lax.rsqrt(var_n + BN_EPS)
    shift_n = bx_ref[...] - m * scale_n
    scale4 = jnp.concatenate([scale_n] * GROUP, axis=1)
    shift4 = jnp.concatenate([shift_n] * GROUP, axis=1)
    xn = xg * scale4 + shift4
    agt_ref[...] = _dot_ta_tb(w4s_ref[...], xn)
    bgt_ref[...] = _dot_ta_tb(w4d_ref[...], xn)


def _build_tables(partials, xg, gx, bx, ge, be, w4s, w4d, w1et, b1, n_edges):
    n_groups = xg.shape[0]
    n_nf = gx.shape[1]
    n_ef = w1et.shape[1]
    args = [partials, xg, gx, bx, ge, be, w4s, w4d, w1et, b1]
    return pl.pallas_call(
        functools.partial(_table_kernel, n_edges=n_edges, n_nf=n_nf),
        grid=(1,),
        in_specs=[pl.BlockSpec(a.shape, lambda t, n=len(a.shape): (0,) * n)
                  for a in args],
        out_specs=[
            pl.BlockSpec((GROUP * 64, n_groups), lambda t: (0, 0)),
            pl.BlockSpec((GROUP * 64, n_groups), lambda t: (0, 0)),
            pl.BlockSpec((64, n_ef), lambda t: (0, 0)),
            pl.BlockSpec((64, HALF_E), lambda t: (0, 0)),
        ],
        out_shape=[
            jax.ShapeDtypeStruct((GROUP * 64, n_groups), jnp.float32),
            jax.ShapeDtypeStruct((GROUP * 64, n_groups), jnp.float32),
            jax.ShapeDtypeStruct((64, n_ef), jnp.float32),
            jax.ShapeDtypeStruct((64, HALF_E), jnp.float32),
        ],
        compiler_params=pltpu.CompilerParams(
            dimension_semantics=("arbitrary",)),
    )(*args)


def _main_kernel(agt_ref, bgt_ref, w1ee_ref, b1cb_ref,
                 w2a_ref, w3a_ref, w4a_ref, w5a_ref, w6a_ref,
                 e_ref, idx_ref, out_ref, *, n_groups):
    agt = agt_ref[...]
    bgt = bgt_ref[...]
    w1ee = w1ee_ref[...]
    b1cb = b1cb_ref[...]
    ones_row = jnp.ones((1, HALF_E), jnp.float32)
    iota_g = lax.broadcasted_iota(jnp.int32, (n_groups, HALF_E), 0)
    n_half = TILE_E // HALF_E
    slices = [slice(h * HALF_E, (h + 1) * HALF_E) for h in range(n_half)]

    hs = []
    for sl in slices:
        src = idx_ref[0:1, sl]
        dst = idx_ref[1:2, sl]
        ms = jnp.where(iota_g == (src >> 2), 1.0, 0.0)
        md = jnp.where(iota_g == (dst >> 2), 1.0, 0.0)
        gs = _dot(agt, ms)
        gd = _dot(bgt, md)
        srcr = src & 3
        dstr = dst & 3
        h1 = b1cb + _dot_tb(w1ee, e_ref[sl, :])
        for k in range(GROUP):
            fs = jnp.where(srcr == k, 1.0, 0.0)
            fd = jnp.where(dstr == k, 1.0, 0.0)
            h1 = h1 + fs * gs[k * 64:(k + 1) * 64, :]
            h1 = h1 + fd * gd[k * 64:(k + 1) * 64, :]
        hs.append(_leaky(h1))
    for wa_ref in (w2a_ref, w3a_ref, w4a_ref, w5a_ref):
        wa = wa_ref[...]
        hs = [_leaky(_dot(wa, jnp.concatenate([h, ones_row], axis=0)))
              for h in hs]
    w6a = w6a_ref[...]
    for h, sl in zip(hs, slices):
        out8 = _dot(w6a, jnp.concatenate([h, ones_row], axis=0))
        out_ref[:, sl] = out8[0:2, :]


def _edge_mlp(agt, bgt, w1ee, b1cb, tails, e_pad, idx_pad, n_tiles):
    n_groups = agt.shape[1]
    n_ef = e_pad.shape[1]
    e_rows = e_pad.shape[0]
    small = [agt, bgt, w1ee, b1cb] + list(tails)
    in_specs = (
        [pl.BlockSpec(a.shape, lambda t: (0, 0)) for a in small]
        + [pl.BlockSpec((TILE_E, n_ef), lambda t: (t, 0)),
           pl.BlockSpec((2, TILE_E), lambda t: (0, t))]
    )
    mlp_flops = 2 * (64 * (2 * GROUP * 64 + n_ef) + 64 * 65 + 32 * 65
                     + 16 * 33 + 8 * 17 + 8 * 9)
    cost = pl.CostEstimate(
        flops=e_rows * mlp_flops + 2 * e_rows * n_groups * GROUP * 64,
        transcendentals=0,
        bytes_accessed=4 * e_rows * (n_ef + 4),
    )
    return pl.pallas_call(
        functools.partial(_main_kernel, n_groups=n_groups),
        grid=(n_tiles,),
        in_specs=in_specs,
        out_specs=pl.BlockSpec((2, TILE_E), lambda t: (0, t)),
        out_shape=jax.ShapeDtypeStruct((2, e_rows), jnp.float32),
        compiler_params=pltpu.CompilerParams(
            dimension_semantics=("parallel",)),
        cost_estimate=cost,
    )(*small, e_pad, idx_pad)


def kernel(x, edge_index, e, xbatch,
           bn_node_gamma, bn_node_beta, bn_edge_gamma, bn_edge_beta,
           w1, b1, w2, b2, w3, b3, w4, b4, w5, b5, w6, b6):
    del xbatch
    f32 = jnp.float32
    n_nodes, n_nf = x.shape
    n_edges, n_ef = e.shape
    n_groups = n_nodes // GROUP

    e_rows = _round_up(max(n_edges, 1), max(TILE_E, TILE_P))
    e_pad = jnp.pad(e.astype(f32), ((0, e_rows - n_edges), (0, 0)))
    idx_pad = jnp.pad(edge_index.astype(jnp.int32),
                      ((0, 0), (0, e_rows - n_edges)))

    xg = x.astype(f32).reshape(n_groups, GROUP * n_nf)
    w1s, w1d = w1[:n_nf], w1[n_nf:2 * n_nf]
    eye4 = jnp.eye(GROUP, dtype=f32)
    w4s = jnp.kron(eye4, w1s)
    w4d = jnp.kron(eye4, w1d)
    w1et = w1[2 * n_nf:].T

    def aug(w, b):
        return jnp.concatenate([w.T, b.reshape(1, -1).T], axis=1)

    w6a = jnp.zeros((8, w6.shape[0] + 1), f32).at[:w6.shape[1]].set(aug(w6, b6))
    tails = (aug(w2, b2), aug(w3, b3), aug(w4, b4), aug(w5, b5), w6a)

    partials = _edge_stats(e_pad, n_ef, e_rows // TILE_P)
    agt, bgt, w1ee, b1cb = _build_tables(
        partials, xg,
        bn_node_gamma.reshape(1, -1).astype(f32),
        bn_node_beta.reshape(1, -1).astype(f32),
        bn_edge_gamma.reshape(1, -1).astype(f32),
        bn_edge_beta.reshape(1, -1).astype(f32),
        w4s, w4d, w1et, b1.reshape(1, -1).astype(f32), n_edges)
    out2 = _edge_mlp(agt, bgt, w1ee, b1cb, tails, e_pad, idx_pad,
                     e_rows // TILE_E)
    return {'edge_pred': [out2[:, :n_edges].T]}


# speedup vs baseline: 4.5431x; 1.0127x over previous
import functools
import jax
import jax.numpy as jnp
from jax import lax
from jax.experimental import pallas as pl
from jax.experimental.pallas import tpu as pltpu

LEAK = 0.1
BN_EPS = 1e-5
TILE_E = 8192
HALF_E = 512
TILE_P = 8192
GROUP = 4
BF = jnp.bfloat16


def _round_up(a, b):
    return (a + b - 1) // b * b


def _dot(a, b):
    return jnp.dot(a, b, preferred_element_type=jnp.float32)


def _dot_tb(a, b):
    return lax.dot_general(a, b, (((1,), (1,)), ((), ())),
                           preferred_element_type=jnp.float32)


def _dot_ta_tb(a, b):
    return lax.dot_general(a, b, (((0,), (1,)), ((), ())),
                           preferred_element_type=jnp.float32)


def _leaky(h):
    return jnp.maximum(h, h * LEAK)


def _eye(n):
    r = lax.broadcasted_iota(jnp.int32, (n, n), 0)
    c = lax.broadcasted_iota(jnp.int32, (n, n), 1)
    return jnp.where(r == c, 1.0, 0.0)


def _outer(row, width):
    ones = jnp.ones((1, width), jnp.float32)
    return lax.dot_general(row, ones, (((0,), (0,)), ((), ())),
                           preferred_element_type=jnp.float32)


def _stats_kernel(e_ref, out_ref):
    e = e_ref[...]
    s = jnp.sum(e, axis=0, keepdims=True)
    ss = jnp.sum(e * e, axis=0, keepdims=True)
    out_ref[...] = jnp.concatenate([s, ss], axis=0).reshape(1, 2, -1)


def _edge_stats(e_pad, n_ef, n_tp):
    return pl.pallas_call(
        _stats_kernel,
        grid=(n_tp,),
        in_specs=[pl.BlockSpec((TILE_P, n_ef), lambda t: (t, 0))],
        out_specs=pl.BlockSpec((1, 2, n_ef), lambda t: (t, 0, 0)),
        out_shape=jax.ShapeDtypeStruct((n_tp, 2, n_ef), jnp.float32),
        compiler_params=pltpu.CompilerParams(
            dimension_semantics=("parallel",)),
    )(e_pad)


def _table_kernel(part_ref, xg_ref, gx_ref, bx_ref, ge_ref, be_ref,
                  w1_ref, b1_ref, w2_ref, b2_ref, w3_ref, b3_ref,
                  w4_ref, b4_ref, w5_ref, b5_ref, w6_ref, b6_ref,
                  agt_ref, bgt_ref, w1ee_ref, b1cb_ref,
                  w2t_ref, b2cb_ref, w3t_ref, b3cb_ref,
                  w4t_ref, b4cb_ref, w5t_ref, b5cb_ref,
                  w6t_ref, b6cb_ref,
                  *, n_edges, n_nf):
    w1e = w1_ref[2 * n_nf:, :]
    s = jnp.sum(part_ref[...], axis=0)
    inv_n = jnp.float32(1.0 / n_edges)
    mean_e = s[0:1, :] * inv_n
    var_e = s[1:2, :] * inv_n - mean_e * mean_e
    scale_e = ge_ref[...] * lax.rsqrt(var_e + BN_EPS)
    shift_e = be_ref[...] - mean_e * scale_e
    w1et = _dot_ta_tb(w1e, _eye(n_nf))
    w1ee_ref[...] = (w1et * scale_e).astype(BF)
    b1_eff = b1_ref[...] + _dot_tb(shift_e, w1et)
    b1cb_ref[...] = _outer(b1_eff, HALF_E)

    xg = xg_ref[...]
    s4 = jnp.mean(xg, axis=0, keepdims=True)
    ss4 = jnp.mean(xg * xg, axis=0, keepdims=True)
    m = jnp.zeros((1, n_nf), jnp.float32)
    msq = jnp.zeros((1, n_nf), jnp.float32)
    for k in range(GROUP):
        m = m + s4[:, k * n_nf:(k + 1) * n_nf]
        msq = msq + ss4[:, k * n_nf:(k + 1) * n_nf]
    m = m * (1.0 / GROUP)
    msq = msq * (1.0 / GROUP)
    var_n = msq - m * m
    scale_n = gx_ref[...] * lax.rsqrt(var_n + BN_EPS)
    shift_n = bx_ref[...] - m * scale_n
    scale4 = jnp.concatenate([scale_n] * GROUP, axis=1)
    shift4 = jnp.concatenate([shift_n] * GROUP, axis=1)
    xn = xg * scale4 + shift4
    w1s = w1_ref[0:n_nf, :]
    w1d = w1_ref[n_nf:2 * n_nf, :]
    for k in range(GROUP):
        xk = xn[:, k * n_nf:(k + 1) * n_nf]
        agt_ref[k * 64:(k + 1) * 64, :] = _dot_ta_tb(w1s, xk).astype(BF)
        bgt_ref[k * 64:(k + 1) * 64, :] = _dot_ta_tb(w1d, xk).astype(BF)

    for w_ref, b_ref, wt_ref, bcb_ref in (
            (w2_ref, b2_ref, w2t_ref, b2cb_ref),
            (w3_ref, b3_ref, w3t_ref, b3cb_ref),
            (w4_ref, b4_ref, w4t_ref, b4cb_ref),
            (w5_ref, b5_ref, w5t_ref, b5cb_ref)):
        w = w_ref[...]
        wt_ref[...] = _dot_tb(_eye(w.shape[1]), w).astype(BF)
        bcb_ref[...] = _outer(b_ref[...], HALF_E)
    w6 = w6_ref[...]
    w6t = _dot_tb(_eye(2), w6)
    w6t_ref[...] = jnp.concatenate(
        [w6t, jnp.zeros((6, 8), jnp.float32)], axis=0).astype(BF)
    b6cb_ref[...] = _outer(
        jnp.concatenate([b6_ref[...], jnp.zeros((1, 6), jnp.float32)],
                        axis=1), HALF_E)


def _build_tables(partials, xg, gx, bx, ge, be, ws, n_edges):
    n_groups = xg.shape[0]
    n_nf = gx.shape[1]
    n_ef = ws[0].shape[0] - 2 * n_nf
    args = [partials, xg, gx, bx, ge, be] + list(ws)
    outs = [
        ((GROUP * 64, n_groups), BF),
        ((GROUP * 64, n_groups), BF),
        ((64, n_ef), BF),
        ((64, HALF_E), jnp.float32),
        ((64, 64), BF), ((64, HALF_E), jnp.float32),
        ((32, 64), BF), ((32, HALF_E), jnp.float32),
        ((16, 32), BF), ((16, HALF_E), jnp.float32),
        ((8, 16), BF), ((8, HALF_E), jnp.float32),
        ((8, 8), BF), ((8, HALF_E), jnp.float32),
    ]
    return pl.pallas_call(
        functools.partial(_table_kernel, n_edges=n_edges, n_nf=n_nf),
        grid=(1,),
        in_specs=[pl.BlockSpec(a.shape, lambda t, n=len(a.shape): (0,) * n)
                  for a in args],
        out_specs=[pl.BlockSpec(s, lambda t: (0, 0)) for s, _ in outs],
        out_shape=[jax.ShapeDtypeStruct(s, d) for s, d in outs],
        compiler_params=pltpu.CompilerParams(
            dimension_semantics=("arbitrary",)),
    )(*args)


def _main_kernel(agt_ref, bgt_ref, w1ee_ref, b1cb_ref,
                 w2t_ref, b2cb_ref, w3t_ref, b3cb_ref,
                 w4t_ref, b4cb_ref, w5t_ref, b5cb_ref,
                 w6t_ref, b6cb_ref,
                 e_ref, idx_ref, out_ref, *, n_groups):
    agt = agt_ref[...]
    bgt = bgt_ref[...]
    w1ee = w1ee_ref[...]
    b1cb = b1cb_ref[...]
    iota_g = lax.broadcasted_iota(jnp.int32, (n_groups, HALF_E), 0)
    n_half = TILE_E // HALF_E
    slices = [slice(h * HALF_E, (h + 1) * HALF_E) for h in range(n_half)]

    hs = []
    for sl in slices:
        src = idx_ref[0:1, sl]
        dst = idx_ref[1:2, sl]
        ms = jnp.where(iota_g == (src >> 2), 1.0, 0.0).astype(BF)
        md = jnp.where(iota_g == (dst >> 2), 1.0, 0.0).astype(BF)
        gs = _dot(agt, ms)
        gd = _dot(bgt, md)
        srcr = src & 3
        dstr = dst & 3
        h1 = b1cb + _dot_tb(w1ee, e_ref[sl, :].astype(BF))
        for k in range(GROUP):
            fs = jnp.where(srcr == k, 1.0, 0.0)
            fd = jnp.where(dstr == k, 1.0, 0.0)
            h1 = h1 + fs * gs[k * 64:(k + 1) * 64, :]
            h1 = h1 + fd * gd[k * 64:(k + 1) * 64, :]
        hs.append(_leaky(h1).astype(BF))
    for wt_ref, bcb_ref in ((w2t_ref, b2cb_ref), (w3t_ref, b3cb_ref),
                            (w4t_ref, b4cb_ref), (w5t_ref, b5cb_ref)):
        wt = wt_ref[...]
        bcb = bcb_ref[...]
        hs = [_leaky(_dot(wt, h) + bcb).astype(BF) for h in hs]
    w6t = w6t_ref[...]
    b6cb = b6cb_ref[...]
    for h, sl in zip(hs, slices):
        out8 = _dot(w6t, h) + b6cb
        out_ref[:, sl] = out8[0:2, :]


def _edge_mlp(tables, e_pad, idx_pad, n_tiles):
    n_groups = tables[0].shape[1]
    n_ef = e_pad.shape[1]
    e_rows = e_pad.shape[0]
    in_specs = (
        [pl.BlockSpec(a.shape, lambda t: (0, 0)) for a in tables]
        + [pl.BlockSpec((TILE_E, n_ef), lambda t: (t, 0)),
           pl.BlockSpec((2, TILE_E), lambda t: (0, t))]
    )
    mlp_flops = 2 * (64 * (2 * GROUP * 64 + n_ef) + 64 * 64 + 32 * 64
                     + 16 * 32 + 8 * 16 + 8 * 8)
    cost = pl.CostEstimate(
        flops=e_rows * mlp_flops + 2 * e_rows * n_groups * GROUP * 64,
        transcendentals=0,
        bytes_accessed=4 * e_rows * (n_ef + 4),
    )
    return pl.pallas_call(
        functools.partial(_main_kernel, n_groups=n_groups),
        grid=(n_tiles,),
        in_specs=in_specs,
        out_specs=pl.BlockSpec((2, TILE_E), lambda t: (0, t)),
        out_shape=jax.ShapeDtypeStruct((2, e_rows), jnp.float32),
        compiler_params=pltpu.CompilerParams(
            dimension_semantics=("parallel",)),
        cost_estimate=cost,
    )(*tables, e_pad, idx_pad)


def kernel(x, edge_index, e, xbatch,
           bn_node_gamma, bn_node_beta, bn_edge_gamma, bn_edge_beta,
           w1, b1, w2, b2, w3, b3, w4, b4, w5, b5, w6, b6):
    del xbatch
    f32 = jnp.float32
    n_nodes, n_nf = x.shape
    n_edges, n_ef = e.shape
    n_groups = n_nodes // GROUP

    e_rows = _round_up(max(n_edges, 1), max(TILE_E, TILE_P))
    e_pad = jnp.pad(e.astype(f32), ((0, e_rows - n_edges), (0, 0)))
    idx_pad = jnp.pad(edge_index.astype(jnp.int32),
                      ((0, 0), (0, e_rows - n_edges)))

    xg = x.astype(f32).reshape(n_groups, GROUP * n_nf)

    partials = _edge_stats(e_pad, n_ef, e_rows // TILE_P)
    ws = [w1.astype(f32), b1.reshape(1, -1).astype(f32),
          w2.astype(f32), b2.reshape(1, -1).astype(f32),
          w3.astype(f32), b3.reshape(1, -1).astype(f32),
          w4.astype(f32), b4.reshape(1, -1).astype(f32),
          w5.astype(f32), b5.reshape(1, -1).astype(f32),
          w6.astype(f32), b6.reshape(1, -1).astype(f32)]
    tables = _build_tables(
        partials, xg,
        bn_node_gamma.reshape(1, -1).astype(f32),
        bn_node_beta.reshape(1, -1).astype(f32),
        bn_edge_gamma.reshape(1, -1).astype(f32),
        bn_edge_beta.reshape(1, -1).astype(f32),
        ws, n_edges)
    out2 = _edge_mlp(tables, e_pad, idx_pad, e_rows // TILE_E)
    return {'edge_pred': [out2[:, :n_edges].T]}


# speedup vs baseline: 4.5811x; 1.0084x over previous
import functools
import jax
import jax.numpy as jnp
from jax import lax
from jax.experimental import pallas as pl
from jax.experimental.pallas import tpu as pltpu

LEAK = 0.1
BN_EPS = 1e-5
TILE_E = 16384
HALF_E = 512
TILE_P = 16384
GROUP = 4
BF = jnp.bfloat16


def _round_up(a, b):
    return (a + b - 1) // b * b


def _dot(a, b):
    return jnp.dot(a, b, preferred_element_type=jnp.float32)


def _dot_tb(a, b):
    return lax.dot_general(a, b, (((1,), (1,)), ((), ())),
                           preferred_element_type=jnp.float32)


def _dot_ta_tb(a, b):
    return lax.dot_general(a, b, (((0,), (1,)), ((), ())),
                           preferred_element_type=jnp.float32)


def _leaky(h):
    return jnp.maximum(h, h * LEAK)


def _eye(n):
    r = lax.broadcasted_iota(jnp.int32, (n, n), 0)
    c = lax.broadcasted_iota(jnp.int32, (n, n), 1)
    return jnp.where(r == c, 1.0, 0.0)


def _outer(row, width):
    ones = jnp.ones((1, width), jnp.float32)
    return lax.dot_general(row, ones, (((0,), (0,)), ((), ())),
                           preferred_element_type=jnp.float32)


def _stats_kernel(e_ref, out_ref):
    e = e_ref[...]
    s = jnp.sum(e, axis=0, keepdims=True)
    ss = jnp.sum(e * e, axis=0, keepdims=True)
    out_ref[...] = jnp.concatenate([s, ss], axis=0).reshape(1, 2, -1)


def _edge_stats(e_pad, n_ef, n_tp):
    return pl.pallas_call(
        _stats_kernel,
        grid=(n_tp,),
        in_specs=[pl.BlockSpec((TILE_P, n_ef), lambda t: (t, 0))],
        out_specs=pl.BlockSpec((1, 2, n_ef), lambda t: (t, 0, 0)),
        out_shape=jax.ShapeDtypeStruct((n_tp, 2, n_ef), jnp.float32),
        compiler_params=pltpu.CompilerParams(
            dimension_semantics=("parallel",)),
    )(e_pad)


def _table_kernel(part_ref, xg_ref, gx_ref, bx_ref, ge_ref, be_ref,
                  w1_ref, b1_ref, w2_ref, b2_ref, w3_ref, b3_ref,
                  w4_ref, b4_ref, w5_ref, b5_ref, w6_ref, b6_ref,
                  agt_ref, bgt_ref, w1ee_ref, b1cb_ref,
                  w2t_ref, b2cb_ref, w3t_ref, b3cb_ref,
                  w4t_ref, b4cb_ref, w5t_ref, b5cb_ref,
                  w6t_ref, b6cb_ref,
                  *, n_edges, n_nf):
    w1e = w1_ref[2 * n_nf:, :]
    s = jnp.sum(part_ref[...], axis=0)
    inv_n = jnp.float32(1.0 / n_edges)
    mean_e = s[0:1, :] * inv_n
    var_e = s[1:2, :] * inv_n - mean_e * mean_e
    scale_e = ge_ref[...] * lax.rsqrt(var_e + BN_EPS)
    shift_e = be_ref[...] - mean_e * scale_e
    w1et = _dot_ta_tb(w1e, _eye(n_nf))
    w1ee_ref[...] = (w1et * scale_e).astype(BF)
    b1_eff = b1_ref[...] + _dot_tb(shift_e, w1et)
    b1cb_ref[...] = _outer(b1_eff, HALF_E)

    xg = xg_ref[...]
    s4 = jnp.mean(xg, axis=0, keepdims=True)
    ss4 = jnp.mean(xg * xg, axis=0, keepdims=True)
    m = jnp.zeros((1, n_nf), jnp.float32)
    msq = jnp.zeros((1, n_nf), jnp.float32)
    for k in range(GROUP):
        m = m + s4[:, k * n_nf:(k + 1) * n_nf]
        msq = msq + ss4[:, k * n_nf:(k + 1) * n_nf]
    m = m * (1.0 / GROUP)
    msq = msq * (1.0 / GROUP)
    var_n = msq - m * m
    scale_n = gx_ref[...] * lax.rsqrt(var_n + BN_EPS)
    shift_n = bx_ref[...] - m * scale_n
    scale4 = jnp.concatenate([scale_n] * GROUP, axis=1)
    shift4 = jnp.concatenate([shift_n] * GROUP, axis=1)
    xn = xg * scale4 + shift4
    w1s = w1_ref[0:n_nf, :]
    w1d = w1_ref[n_nf:2 * n_nf, :]
    for k in range(GROUP):
        xk = xn[:, k * n_nf:(k + 1) * n_nf]
        agt_ref[k * 64:(k + 1) * 64, :] = _dot_ta_tb(w1s, xk).astype(BF)
        bgt_ref[k * 64:(k + 1) * 64, :] = _dot_ta_tb(w1d, xk).astype(BF)

    for w_ref, b_ref, wt_ref, bcb_ref in (
            (w2_ref, b2_ref, w2t_ref, b2cb_ref),
            (w3_ref, b3_ref, w3t_ref, b3cb_ref),
            (w4_ref, b4_ref, w4t_ref, b4cb_ref),
            (w5_ref, b5_ref, w5t_ref, b5cb_ref)):
        w = w_ref[...]
        wt_ref[...] = _dot_tb(_eye(w.shape[1]), w).astype(BF)
        bcb_ref[...] = _outer(b_ref[...], HALF_E)
    w6 = w6_ref[...]
    w6t = _dot_tb(_eye(2), w6)
    w6t_ref[...] = jnp.concatenate(
        [w6t, jnp.zeros((6, 8), jnp.float32)], axis=0).astype(BF)
    b6cb_ref[...] = _outer(
        jnp.concatenate([b6_ref[...], jnp.zeros((1, 6), jnp.float32)],
                        axis=1), HALF_E)


def _build_tables(partials, xg, gx, bx, ge, be, ws, n_edges):
    n_groups = xg.shape[0]
    n_nf = gx.shape[1]
    n_ef = ws[0].shape[0] - 2 * n_nf
    args = [partials, xg, gx, bx, ge, be] + list(ws)
    outs = [
        ((GROUP * 64, n_groups), BF),
        ((GROUP * 64, n_groups), BF),
        ((64, n_ef), BF),
        ((64, HALF_E), jnp.float32),
        ((64, 64), BF), ((64, HALF_E), jnp.float32),
        ((32, 64), BF), ((32, HALF_E), jnp.float32),
        ((16, 32), BF), ((16, HALF_E), jnp.float32),
        ((8, 16), BF), ((8, HALF_E), jnp.float32),
        ((8, 8), BF), ((8, HALF_E), jnp.float32),
    ]
    return pl.pallas_call(
        functools.partial(_table_kernel, n_edges=n_edges, n_nf=n_nf),
        grid=(1,),
        in_specs=[pl.BlockSpec(a.shape, lambda t, n=len(a.shape): (0,) * n)
                  for a in args],
        out_specs=[pl.BlockSpec(s, lambda t: (0, 0)) for s, _ in outs],
        out_shape=[jax.ShapeDtypeStruct(s, d) for s, d in outs],
        compiler_params=pltpu.CompilerParams(
            dimension_semantics=("arbitrary",)),
    )(*args)


def _main_kernel(agt_ref, bgt_ref, w1ee_ref, b1cb_ref,
                 w2t_ref, b2cb_ref, w3t_ref, b3cb_ref,
                 w4t_ref, b4cb_ref, w5t_ref, b5cb_ref,
                 w6t_ref, b6cb_ref,
                 e_ref, idx_ref, out_ref, *, n_groups):
    agt = agt_ref[...]
    bgt = bgt_ref[...]
    w1ee = w1ee_ref[...]
    b1cb = b1cb_ref[...]
    iota_g = lax.broadcasted_iota(jnp.int32, (n_groups, HALF_E), 0)
    n_half = TILE_E // HALF_E
    slices = [slice(h * HALF_E, (h + 1) * HALF_E) for h in range(n_half)]

    hs = []
    for sl in slices:
        src = idx_ref[0:1, sl]
        dst = idx_ref[1:2, sl]
        ms = jnp.where(iota_g == (src >> 2), 1.0, 0.0).astype(BF)
        md = jnp.where(iota_g == (dst >> 2), 1.0, 0.0).astype(BF)
        gs = _dot(agt, ms)
        gd = _dot(bgt, md)
        srcr = src & 3
        dstr = dst & 3
        h1 = b1cb + _dot_tb(w1ee, e_ref[sl, :].astype(BF))
        for k in range(GROUP):
            fs = jnp.where(srcr == k, 1.0, 0.0)
            fd = jnp.where(dstr == k, 1.0, 0.0)
            h1 = h1 + fs * gs[k * 64:(k + 1) * 64, :]
            h1 = h1 + fd * gd[k * 64:(k + 1) * 64, :]
        hs.append(_leaky(h1).astype(BF))
    for wt_ref, bcb_ref in ((w2t_ref, b2cb_ref), (w3t_ref, b3cb_ref),
                            (w4t_ref, b4cb_ref), (w5t_ref, b5cb_ref)):
        wt = wt_ref[...]
        bcb = bcb_ref[...]
        hs = [_leaky(_dot(wt, h) + bcb).astype(BF) for h in hs]
    w6t = w6t_ref[...]
    b6cb = b6cb_ref[...]
    for h, sl in zip(hs, slices):
        out8 = _dot(w6t, h) + b6cb
        out_ref[:, sl] = out8[0:2, :]


def _edge_mlp(tables, e_pad, idx_pad, n_tiles):
    n_groups = tables[0].shape[1]
    n_ef = e_pad.shape[1]
    e_rows = e_pad.shape[0]
    in_specs = (
        [pl.BlockSpec(a.shape, lambda t: (0, 0)) for a in tables]
        + [pl.BlockSpec((TILE_E, n_ef), lambda t: (t, 0)),
           pl.BlockSpec((2, TILE_E), lambda t: (0, t))]
    )
    mlp_flops = 2 * (64 * (2 * GROUP * 64 + n_ef) + 64 * 64 + 32 * 64
                     + 16 * 32 + 8 * 16 + 8 * 8)
    cost = pl.CostEstimate(
        flops=e_rows * mlp_flops + 2 * e_rows * n_groups * GROUP * 64,
        transcendentals=0,
        bytes_accessed=4 * e_rows * (n_ef + 4),
    )
    return pl.pallas_call(
        functools.partial(_main_kernel, n_groups=n_groups),
        grid=(n_tiles,),
        in_specs=in_specs,
        out_specs=pl.BlockSpec((2, TILE_E), lambda t: (0, t)),
        out_shape=jax.ShapeDtypeStruct((2, e_rows), jnp.float32),
        compiler_params=pltpu.CompilerParams(
            dimension_semantics=("parallel",)),
        cost_estimate=cost,
    )(*tables, e_pad, idx_pad)


def kernel(x, edge_index, e, xbatch,
           bn_node_gamma, bn_node_beta, bn_edge_gamma, bn_edge_beta,
           w1, b1, w2, b2, w3, b3, w4, b4, w5, b5, w6, b6):
    del xbatch
    f32 = jnp.float32
    n_nodes, n_nf = x.shape
    n_edges, n_ef = e.shape
    n_groups = n_nodes // GROUP

    e_rows = _round_up(max(n_edges, 1), max(TILE_E, TILE_P))
    e_pad = jnp.pad(e.astype(f32), ((0, e_rows - n_edges), (0, 0)))
    idx_pad = jnp.pad(edge_index.astype(jnp.int32),
                      ((0, 0), (0, e_rows - n_edges)))

    xg = x.astype(f32).reshape(n_groups, GROUP * n_nf)

    partials = _edge_stats(e_pad, n_ef, e_rows // TILE_P)
    ws = [w1.astype(f32), b1.reshape(1, -1).astype(f32),
          w2.astype(f32), b2.reshape(1, -1).astype(f32),
          w3.astype(f32), b3.reshape(1, -1).astype(f32),
          w4.astype(f32), b4.reshape(1, -1).astype(f32),
          w5.astype(f32), b5.reshape(1, -1).astype(f32),
          w6.astype(f32), b6.reshape(1, -1).astype(f32)]
    tables = _build_tables(
        partials, xg,
        bn_node_gamma.reshape(1, -1).astype(f32),
        bn_node_beta.reshape(1, -1).astype(f32),
        bn_edge_gamma.reshape(1, -1).astype(f32),
        bn_edge_beta.reshape(1, -1).astype(f32),
        ws, n_edges)
    out2 = _edge_mlp(tables, e_pad, idx_pad, e_rows // TILE_E)
    return {'edge_pred': [out2[:, :n_edges].T]}
